# baseline (speedup 1.0000x reference)
import functools

import jax
import jax.numpy as jnp
from jax.experimental import pallas as pl
from jax.experimental.pallas import tpu as pltpu

STRIDE = 320
HID = 512
CB = 1024
NQ = 4


def _dot_t(a, b):
    return jax.lax.dot_general(a, b, (((1,), (1,)), ((), ())),
                               preferred_element_type=jnp.float32)


def _encvq_kernel(x_ref, w_ref, b_ref, g_ref, beta_ref, cb_ref, q_ref,
                  loss_ref, *, nblk, scale):
    i = pl.program_id(0)
    y = _dot_t(x_ref[...], w_ref[...]) + b_ref[...]
    m = jnp.mean(y, axis=-1, keepdims=True)
    v = jnp.mean((y - m) ** 2, axis=-1, keepdims=True)
    yn = (y - m) * jax.lax.rsqrt(v + 1e-5) * g_ref[...] + beta_ref[...]
    res = jnp.maximum(yn, 0.0)

    quant = jnp.zeros_like(res)
    loss = jnp.float32(0.0)
    for qi in range(NQ):
        cb = cb_ref[qi]
        d = (jnp.sum(res * res, axis=-1, keepdims=True)
             - 2.0 * _dot_t(res, cb)
             + jnp.sum(cb * cb, axis=-1)[None, :])
        idx = jnp.argmin(d, axis=-1)
        onehot = (jax.lax.broadcasted_iota(jnp.int32, d.shape, 1)
                  == idx[:, None]).astype(jnp.float32)
        q = jax.lax.dot_general(onehot, cb, (((1,), (0,)), ((), ())),
                                preferred_element_type=jnp.float32,
                                precision=jax.lax.Precision.HIGHEST)
        loss = loss + jnp.sum((q - res) ** 2)
        res = res - q
        quant = quant + q
    q_ref[...] = quant

    lv = loss.reshape(1, 1)

    @pl.when(i == 0)
    def _init():
        loss_ref[...] = lv

    @pl.when(i > 0)
    def _acc():
        loss_ref[...] += lv

    @pl.when(i == nblk - 1)
    def _fin():
        loss_ref[...] = loss_ref[...] * scale


def _dec_kernel(x_ref, wih0_ref, whh0_ref, b0_ref, wcat1_ref, b1_ref,
                outw_ref, outb_ref, o_ref,
                xw0_ref, h1buf_ref, h0_ref, c0_ref, h1_ref, c1_ref,
                *, bt, bn):
    i = pl.program_id(0)

    @pl.when(i == 0)
    def _init():
        h0_ref[...] = jnp.zeros_like(h0_ref)
        c0_ref[...] = jnp.zeros_like(c0_ref)
        h1_ref[...] = jnp.zeros_like(h1_ref)
        c1_ref[...] = jnp.zeros_like(c1_ref)

    xw0_ref[...] = _dot_t(x_ref[...], wih0_ref[...]) + b0_ref[...]

    def gates_to_hc(gates, c):
        i_g = jax.nn.sigmoid(gates[:, :HID])
        f_g = jax.nn.sigmoid(gates[:, HID:2 * HID])
        g_g = jnp.tanh(gates[:, 2 * HID:3 * HID])
        o_g = jax.nn.sigmoid(gates[:, 3 * HID:])
        c2 = f_g * c + i_g * g_g
        return o_g * jnp.tanh(c2), c2

    def step(t, _):
        g0 = xw0_ref[pl.ds(t * bn, bn), :] + _dot_t(h0_ref[...],
                                                    whh0_ref[...])
        h0, c0 = gates_to_hc(g0, c0_ref[...])
        h0_ref[...] = h0
        c0_ref[...] = c0
        hcat = jnp.concatenate([h0, h1_ref[...]], axis=1)
        g1 = _dot_t(hcat, wcat1_ref[...]) + b1_ref[...]
        h1, c1 = gates_to_hc(g1, c1_ref[...])
        h1_ref[...] = h1
        c1_ref[...] = c1
        h1buf_ref[pl.ds(t * bn, bn), :] = h1
        return 0

    jax.lax.fori_loop(0, bt, step, 0)

    o_ref[...] = _dot_t(h1buf_ref[...], outw_ref[...]) + outb_ref[...]


def kernel(waveform, enc_W, enc_b, ln_g, ln_b, codebooks, Wih0, Whh0, bih0,
           bhh0, Wih1, Whh1, bih1, bhh1, out_W, out_b):
    Bn, T = waveform.shape
    frames = T // STRIDE
    rows = Bn * frames
    rb = 1000 if rows % 1000 == 0 else rows

    x = waveform.reshape(rows, STRIDE)

    nblk = rows // rb
    scale = 1.0 / (2.0 * rows * HID)
    quant, loss = pl.pallas_call(
        functools.partial(_encvq_kernel, nblk=nblk, scale=scale),
        grid=(nblk,),
        in_specs=[
            pl.BlockSpec((rb, STRIDE), lambda i: (i, 0)),
            pl.BlockSpec((HID, STRIDE), lambda i: (0, 0)),
            pl.BlockSpec((1, HID), lambda i: (0, 0)),
            pl.BlockSpec((1, HID), lambda i: (0, 0)),
            pl.BlockSpec((1, HID), lambda i: (0, 0)),
            pl.BlockSpec((NQ, CB, HID), lambda i: (0, 0, 0)),
        ],
        out_specs=[
            pl.BlockSpec((rb, HID), lambda i: (i, 0)),
            pl.BlockSpec((1, 1), lambda i: (0, 0)),
        ],
        out_shape=[
            jax.ShapeDtypeStruct((rows, HID), jnp.float32),
            jax.ShapeDtypeStruct((1, 1), jnp.float32),
        ],
    )(x, enc_W, enc_b.reshape(1, HID), ln_g.reshape(1, HID),
      ln_b.reshape(1, HID), codebooks)

    dec_in = (quant.reshape(Bn, frames, HID).swapaxes(0, 1)
              .reshape(rows, HID))

    bt_blk = 50 if frames % 50 == 0 else frames
    rbt = bt_blk * Bn
    b0 = (bih0 + bhh0).reshape(1, 4 * HID)
    b1 = (bih1 + bhh1).reshape(1, 4 * HID)
    Wcat1 = jnp.concatenate([Wih1, Whh1], axis=1)

    out_flat = pl.pallas_call(
        functools.partial(_dec_kernel, bt=bt_blk, bn=Bn),
        grid=(frames // bt_blk,),
        in_specs=[
            pl.BlockSpec((rbt, HID), lambda i: (i, 0)),
            pl.BlockSpec((4 * HID, HID), lambda i: (0, 0)),
            pl.BlockSpec((4 * HID, HID), lambda i: (0, 0)),
            pl.BlockSpec((1, 4 * HID), lambda i: (0, 0)),
            pl.BlockSpec((4 * HID, 2 * HID), lambda i: (0, 0)),
            pl.BlockSpec((1, 4 * HID), lambda i: (0, 0)),
            pl.BlockSpec((STRIDE, HID), lambda i: (0, 0)),
            pl.BlockSpec((1, STRIDE), lambda i: (0, 0)),
        ],
        out_specs=pl.BlockSpec((rbt, STRIDE), lambda i: (i, 0)),
        out_shape=jax.ShapeDtypeStruct((rows, STRIDE), jnp.float32),
        scratch_shapes=[
            pltpu.VMEM((rbt, 4 * HID), jnp.float32),
            pltpu.VMEM((rbt, HID), jnp.float32),
            pltpu.VMEM((Bn, HID), jnp.float32),
            pltpu.VMEM((Bn, HID), jnp.float32),
            pltpu.VMEM((Bn, HID), jnp.float32),
            pltpu.VMEM((Bn, HID), jnp.float32),
        ],
    )(dec_in, Wih0, Whh0, b0, Wcat1, b1, out_W, out_b.reshape(1, STRIDE))

    out = (out_flat.reshape(frames, Bn, STRIDE).swapaxes(0, 1)
           .reshape(Bn, frames * STRIDE))
    return out, loss.reshape(())


# speedup vs baseline: 1.1856x; 1.1856x over previous
import functools

import jax
import jax.numpy as jnp
from jax.experimental import pallas as pl
from jax.experimental.pallas import tpu as pltpu

STRIDE = 320
HID = 512
CB = 1024
NQ = 4


def _dot_t(a, b):
    return jax.lax.dot_general(a, b, (((1,), (1,)), ((), ())),
                               preferred_element_type=jnp.float32)


def _encvq_kernel(x_ref, w_ref, b_ref, g_ref, beta_ref, cb_ref, q_ref,
                  loss_ref, *, nblk, scale):
    i = pl.program_id(0)
    y = _dot_t(x_ref[...], w_ref[...]) + b_ref[...]
    m = jnp.mean(y, axis=-1, keepdims=True)
    v = jnp.mean((y - m) ** 2, axis=-1, keepdims=True)
    yn = (y - m) * jax.lax.rsqrt(v + 1e-5) * g_ref[...] + beta_ref[...]
    res = jnp.maximum(yn, 0.0)

    quant = jnp.zeros_like(res)
    loss = jnp.float32(0.0)
    for qi in range(NQ):
        cb = cb_ref[qi]
        d = (jnp.sum(res * res, axis=-1, keepdims=True)
             - 2.0 * _dot_t(res, cb)
             + jnp.sum(cb * cb, axis=-1)[None, :])
        idx = jnp.argmin(d, axis=-1)
        onehot = (jax.lax.broadcasted_iota(jnp.int32, d.shape, 1)
                  == idx[:, None]).astype(jnp.float32)
        q = jax.lax.dot_general(onehot, cb, (((1,), (0,)), ((), ())),
                                preferred_element_type=jnp.float32,
                                precision=jax.lax.Precision.HIGHEST)
        loss = loss + jnp.sum((q - res) ** 2)
        res = res - q
        quant = quant + q
    q_ref[...] = quant

    lv = loss.reshape(1, 1)

    @pl.when(i == 0)
    def _init():
        loss_ref[...] = lv

    @pl.when(i > 0)
    def _acc():
        loss_ref[...] += lv

    @pl.when(i == nblk - 1)
    def _fin():
        loss_ref[...] = loss_ref[...] * scale


def _gates_to_hc(gates, c):
    i_g = jax.nn.sigmoid(gates[:, :HID])
    f_g = jax.nn.sigmoid(gates[:, HID:2 * HID])
    g_g = jnp.tanh(gates[:, 2 * HID:3 * HID])
    o_g = jax.nn.sigmoid(gates[:, 3 * HID:])
    c2 = f_g * c + i_g * g_g
    return o_g * jnp.tanh(c2), c2


def _lstm_kernel(x_ref, wih_ref, whh_ref, b_ref, o_ref, xw_ref, h_ref, c_ref,
                 *, bt, bn):
    i = pl.program_id(0)

    @pl.when(i == 0)
    def _init():
        h_ref[...] = jnp.zeros_like(h_ref)
        c_ref[...] = jnp.zeros_like(c_ref)

    xw_ref[...] = _dot_t(x_ref[...], wih_ref[...]) + b_ref[...]

    def step(t, _):
        gates = xw_ref[pl.ds(t * bn, bn), :] + _dot_t(h_ref[...], whh_ref[...])
        h2, c2 = _gates_to_hc(gates, c_ref[...])
        c_ref[...] = c2
        h_ref[...] = h2
        o_ref[pl.ds(t * bn, bn), :] = h2
        return 0

    jax.lax.fori_loop(0, bt, step, 0)


def _lstm_proj_kernel(x_ref, wih_ref, whh_ref, b_ref, outw_ref, outb_ref,
                      o_ref, xw_ref, hbuf_ref, h_ref, c_ref, *, bt, bn):
    i = pl.program_id(0)

    @pl.when(i == 0)
    def _init():
        h_ref[...] = jnp.zeros_like(h_ref)
        c_ref[...] = jnp.zeros_like(c_ref)

    xw_ref[...] = _dot_t(x_ref[...], wih_ref[...]) + b_ref[...]

    def step(t, _):
        gates = xw_ref[pl.ds(t * bn, bn), :] + _dot_t(h_ref[...], whh_ref[...])
        h2, c2 = _gates_to_hc(gates, c_ref[...])
        c_ref[...] = c2
        h_ref[...] = h2
        hbuf_ref[pl.ds(t * bn, bn), :] = h2
        return 0

    jax.lax.fori_loop(0, bt, step, 0)

    o_ref[...] = _dot_t(hbuf_ref[...], outw_ref[...]) + outb_ref[...]


def kernel(waveform, enc_W, enc_b, ln_g, ln_b, codebooks, Wih0, Whh0, bih0,
           bhh0, Wih1, Whh1, bih1, bhh1, out_W, out_b):
    Bn, T = waveform.shape
    frames = T // STRIDE
    rows = Bn * frames
    rb = 1000 if rows % 1000 == 0 else rows

    x = waveform.reshape(rows, STRIDE)

    nblk = rows // rb
    scale = 1.0 / (2.0 * rows * HID)
    quant, loss = pl.pallas_call(
        functools.partial(_encvq_kernel, nblk=nblk, scale=scale),
        grid=(nblk,),
        in_specs=[
            pl.BlockSpec((rb, STRIDE), lambda i: (i, 0)),
            pl.BlockSpec((HID, STRIDE), lambda i: (0, 0)),
            pl.BlockSpec((1, HID), lambda i: (0, 0)),
            pl.BlockSpec((1, HID), lambda i: (0, 0)),
            pl.BlockSpec((1, HID), lambda i: (0, 0)),
            pl.BlockSpec((NQ, CB, HID), lambda i: (0, 0, 0)),
        ],
        out_specs=[
            pl.BlockSpec((rb, HID), lambda i: (i, 0)),
            pl.BlockSpec((1, 1), lambda i: (0, 0)),
        ],
        out_shape=[
            jax.ShapeDtypeStruct((rows, HID), jnp.float32),
            jax.ShapeDtypeStruct((1, 1), jnp.float32),
        ],
    )(x, enc_W, enc_b.reshape(1, HID), ln_g.reshape(1, HID),
      ln_b.reshape(1, HID), codebooks)

    dec_in = (quant.reshape(Bn, frames, HID).swapaxes(0, 1)
              .reshape(rows, HID))

    bt_blk = 50 if frames % 50 == 0 else frames
    rbt = bt_blk * Bn
    b0 = (bih0 + bhh0).reshape(1, 4 * HID)
    b1 = (bih1 + bhh1).reshape(1, 4 * HID)

    h0 = pl.pallas_call(
        functools.partial(_lstm_kernel, bt=bt_blk, bn=Bn),
        grid=(frames // bt_blk,),
        in_specs=[
            pl.BlockSpec((rbt, HID), lambda i: (i, 0)),
            pl.BlockSpec((4 * HID, HID), lambda i: (0, 0)),
            pl.BlockSpec((4 * HID, HID), lambda i: (0, 0)),
            pl.BlockSpec((1, 4 * HID), lambda i: (0, 0)),
        ],
        out_specs=pl.BlockSpec((rbt, HID), lambda i: (i, 0)),
        out_shape=jax.ShapeDtypeStruct((rows, HID), jnp.float32),
        scratch_shapes=[
            pltpu.VMEM((rbt, 4 * HID), jnp.float32),
            pltpu.VMEM((Bn, HID), jnp.float32),
            pltpu.VMEM((Bn, HID), jnp.float32),
        ],
    )(dec_in, Wih0, Whh0, b0)

    out_flat = pl.pallas_call(
        functools.partial(_lstm_proj_kernel, bt=bt_blk, bn=Bn),
        grid=(frames // bt_blk,),
        in_specs=[
            pl.BlockSpec((rbt, HID), lambda i: (i, 0)),
            pl.BlockSpec((4 * HID, HID), lambda i: (0, 0)),
            pl.BlockSpec((4 * HID, HID), lambda i: (0, 0)),
            pl.BlockSpec((1, 4 * HID), lambda i: (0, 0)),
            pl.BlockSpec((STRIDE, HID), lambda i: (0, 0)),
            pl.BlockSpec((1, STRIDE), lambda i: (0, 0)),
        ],
        out_specs=pl.BlockSpec((rbt, STRIDE), lambda i: (i, 0)),
        out_shape=jax.ShapeDtypeStruct((rows, STRIDE), jnp.float32),
        scratch_shapes=[
            pltpu.VMEM((rbt, 4 * HID), jnp.float32),
            pltpu.VMEM((rbt, HID), jnp.float32),
            pltpu.VMEM((Bn, HID), jnp.float32),
            pltpu.VMEM((Bn, HID), jnp.float32),
        ],
    )(h0, Wih1, Whh1, b1, out_W, out_b.reshape(1, STRIDE))

    out = (out_flat.reshape(frames, Bn, STRIDE).swapaxes(0, 1)
           .reshape(Bn, frames * STRIDE))
    return out, loss.reshape(())


# speedup vs baseline: 1.2425x; 1.0480x over previous
import functools

import jax
import jax.numpy as jnp
from jax.experimental import pallas as pl
from jax.experimental.pallas import tpu as pltpu

STRIDE = 320
HID = 512
CB = 1024
NQ = 4


def _dot_t(a, b):
    return jax.lax.dot_general(a, b, (((1,), (1,)), ((), ())),
                               preferred_element_type=jnp.float32)


def _encvq_kernel(x_ref, w_ref, b_ref, g_ref, beta_ref, cb_ref, q_ref,
                  loss_ref, *, nblk, scale):
    i = pl.program_id(0)
    y = _dot_t(x_ref[...], w_ref[...]) + b_ref[...]
    m = jnp.mean(y, axis=-1, keepdims=True)
    v = jnp.mean((y - m) ** 2, axis=-1, keepdims=True)
    yn = (y - m) * jax.lax.rsqrt(v + 1e-5) * g_ref[...] + beta_ref[...]
    res = jnp.maximum(yn, 0.0)

    quant = jnp.zeros_like(res)
    loss = jnp.float32(0.0)
    for qi in range(NQ):
        cb = cb_ref[qi]
        d = (jnp.sum(res * res, axis=-1, keepdims=True)
             - 2.0 * _dot_t(res, cb)
             + jnp.sum(cb * cb, axis=-1)[None, :])
        idx = jnp.argmin(d, axis=-1)
        onehot = (jax.lax.broadcasted_iota(jnp.int32, d.shape, 1)
                  == idx[:, None]).astype(jnp.float32)
        q = jax.lax.dot_general(onehot, cb, (((1,), (0,)), ((), ())),
                                preferred_element_type=jnp.float32,
                                precision=jax.lax.Precision.HIGHEST)
        loss = loss + jnp.sum((q - res) ** 2)
        res = res - q
        quant = quant + q
    q_ref[...] = quant

    lv = loss.reshape(1, 1)

    @pl.when(i == 0)
    def _init():
        loss_ref[...] = lv

    @pl.when(i > 0)
    def _acc():
        loss_ref[...] += lv

    @pl.when(i == nblk - 1)
    def _fin():
        loss_ref[...] = loss_ref[...] * scale


def _gates_to_hc(gates, c):
    i_g = jax.nn.sigmoid(gates[:, :HID])
    f_g = jax.nn.sigmoid(gates[:, HID:2 * HID])
    g_g = jnp.tanh(gates[:, 2 * HID:3 * HID])
    o_g = jax.nn.sigmoid(gates[:, 3 * HID:])
    c2 = f_g * c + i_g * g_g
    return o_g * jnp.tanh(c2), c2


def _lstm_kernel(x_ref, wih_ref, whh_ref, b_ref, o_ref, xw_ref, h_ref, c_ref,
                 *, bt, bn):
    i = pl.program_id(0)

    @pl.when(i == 0)
    def _init():
        h_ref[...] = jnp.zeros_like(h_ref)
        c_ref[...] = jnp.zeros_like(c_ref)

    xw_ref[...] = _dot_t(x_ref[...], wih_ref[...]) + b_ref[...]

    def step(t, _):
        gates = xw_ref[pl.ds(t * bn, bn), :] + _dot_t(h_ref[...], whh_ref[...])
        h2, c2 = _gates_to_hc(gates, c_ref[...])
        c_ref[...] = c2
        h_ref[...] = h2
        o_ref[pl.ds(t * bn, bn), :] = h2
        return 0

    jax.lax.fori_loop(0, bt, step, 0, unroll=5)


def _lstm_proj_kernel(x_ref, wih_ref, whh_ref, b_ref, outw_ref, outb_ref,
                      o_ref, xw_ref, hbuf_ref, h_ref, c_ref, *, bt, bn):
    i = pl.program_id(0)

    @pl.when(i == 0)
    def _init():
        h_ref[...] = jnp.zeros_like(h_ref)
        c_ref[...] = jnp.zeros_like(c_ref)

    xw_ref[...] = _dot_t(x_ref[...], wih_ref[...]) + b_ref[...]

    def step(t, _):
        gates = xw_ref[pl.ds(t * bn, bn), :] + _dot_t(h_ref[...], whh_ref[...])
        h2, c2 = _gates_to_hc(gates, c_ref[...])
        c_ref[...] = c2
        h_ref[...] = h2
        hbuf_ref[pl.ds(t * bn, bn), :] = h2
        return 0

    jax.lax.fori_loop(0, bt, step, 0, unroll=5)

    o_ref[...] = _dot_t(hbuf_ref[...], outw_ref[...]) + outb_ref[...]


def kernel(waveform, enc_W, enc_b, ln_g, ln_b, codebooks, Wih0, Whh0, bih0,
           bhh0, Wih1, Whh1, bih1, bhh1, out_W, out_b):
    Bn, T = waveform.shape
    frames = T // STRIDE
    rows = Bn * frames
    rb = 1000 if rows % 1000 == 0 else rows

    x = waveform.reshape(rows, STRIDE)

    nblk = rows // rb
    scale = 1.0 / (2.0 * rows * HID)
    quant, loss = pl.pallas_call(
        functools.partial(_encvq_kernel, nblk=nblk, scale=scale),
        grid=(nblk,),
        in_specs=[
            pl.BlockSpec((rb, STRIDE), lambda i: (i, 0)),
            pl.BlockSpec((HID, STRIDE), lambda i: (0, 0)),
            pl.BlockSpec((1, HID), lambda i: (0, 0)),
            pl.BlockSpec((1, HID), lambda i: (0, 0)),
            pl.BlockSpec((1, HID), lambda i: (0, 0)),
            pl.BlockSpec((NQ, CB, HID), lambda i: (0, 0, 0)),
        ],
        out_specs=[
            pl.BlockSpec((rb, HID), lambda i: (i, 0)),
            pl.BlockSpec((1, 1), lambda i: (0, 0)),
        ],
        out_shape=[
            jax.ShapeDtypeStruct((rows, HID), jnp.float32),
            jax.ShapeDtypeStruct((1, 1), jnp.float32),
        ],
    )(x, enc_W, enc_b.reshape(1, HID), ln_g.reshape(1, HID),
      ln_b.reshape(1, HID), codebooks)

    dec_in = (quant.reshape(Bn, frames, HID).swapaxes(0, 1)
              .reshape(rows, HID))

    bt_blk = 50 if frames % 50 == 0 else frames
    rbt = bt_blk * Bn
    b0 = (bih0 + bhh0).reshape(1, 4 * HID)
    b1 = (bih1 + bhh1).reshape(1, 4 * HID)

    h0 = pl.pallas_call(
        functools.partial(_lstm_kernel, bt=bt_blk, bn=Bn),
        grid=(frames // bt_blk,),
        in_specs=[
            pl.BlockSpec((rbt, HID), lambda i: (i, 0)),
            pl.BlockSpec((4 * HID, HID), lambda i: (0, 0)),
            pl.BlockSpec((4 * HID, HID), lambda i: (0, 0)),
            pl.BlockSpec((1, 4 * HID), lambda i: (0, 0)),
        ],
        out_specs=pl.BlockSpec((rbt, HID), lambda i: (i, 0)),
        out_shape=jax.ShapeDtypeStruct((rows, HID), jnp.float32),
        scratch_shapes=[
            pltpu.VMEM((rbt, 4 * HID), jnp.float32),
            pltpu.VMEM((Bn, HID), jnp.float32),
            pltpu.VMEM((Bn, HID), jnp.float32),
        ],
    )(dec_in, Wih0, Whh0, b0)

    out_flat = pl.pallas_call(
        functools.partial(_lstm_proj_kernel, bt=bt_blk, bn=Bn),
        grid=(frames // bt_blk,),
        in_specs=[
            pl.BlockSpec((rbt, HID), lambda i: (i, 0)),
            pl.BlockSpec((4 * HID, HID), lambda i: (0, 0)),
            pl.BlockSpec((4 * HID, HID), lambda i: (0, 0)),
            pl.BlockSpec((1, 4 * HID), lambda i: (0, 0)),
            pl.BlockSpec((STRIDE, HID), lambda i: (0, 0)),
            pl.BlockSpec((1, STRIDE), lambda i: (0, 0)),
        ],
        out_specs=pl.BlockSpec((rbt, STRIDE), lambda i: (i, 0)),
        out_shape=jax.ShapeDtypeStruct((rows, STRIDE), jnp.float32),
        scratch_shapes=[
            pltpu.VMEM((rbt, 4 * HID), jnp.float32),
            pltpu.VMEM((rbt, HID), jnp.float32),
            pltpu.VMEM((Bn, HID), jnp.float32),
            pltpu.VMEM((Bn, HID), jnp.float32),
        ],
    )(h0, Wih1, Whh1, b1, out_W, out_b.reshape(1, STRIDE))

    out = (out_flat.reshape(frames, Bn, STRIDE).swapaxes(0, 1)
           .reshape(Bn, frames * STRIDE))
    return out, loss.reshape(())


# speedup vs baseline: 1.2545x; 1.0097x over previous
import functools

import jax
import jax.numpy as jnp
from jax.experimental import pallas as pl
from jax.experimental.pallas import tpu as pltpu

STRIDE = 320
HID = 512
CB = 1024
NQ = 4


def _dot_t(a, b):
    return jax.lax.dot_general(a, b, (((1,), (1,)), ((), ())),
                               preferred_element_type=jnp.float32)


def _encvq_kernel(x_ref, w_ref, b_ref, g_ref, beta_ref, cb_ref, q_ref,
                  loss_ref, *, nblk, scale):
    i = pl.program_id(0)
    y = _dot_t(x_ref[...], w_ref[...]) + b_ref[...]
    m = jnp.mean(y, axis=-1, keepdims=True)
    v = jnp.mean((y - m) ** 2, axis=-1, keepdims=True)
    yn = (y - m) * jax.lax.rsqrt(v + 1e-5) * g_ref[...] + beta_ref[...]
    res = jnp.maximum(yn, 0.0)

    quant = jnp.zeros_like(res)
    loss = jnp.float32(0.0)
    for qi in range(NQ):
        cb = cb_ref[qi]
        d = (jnp.sum(res * res, axis=-1, keepdims=True)
             - 2.0 * _dot_t(res, cb)
             + jnp.sum(cb * cb, axis=-1)[None, :])
        idx = jnp.argmin(d, axis=-1)
        onehot = (jax.lax.broadcasted_iota(jnp.int32, d.shape, 1)
                  == idx[:, None]).astype(jnp.float32)
        q = jax.lax.dot_general(onehot, cb, (((1,), (0,)), ((), ())),
                                preferred_element_type=jnp.float32,
                                precision=jax.lax.Precision.HIGHEST)
        loss = loss + jnp.sum((q - res) ** 2)
        res = res - q
        quant = quant + q
    q_ref[...] = quant

    lv = loss.reshape(1, 1)

    @pl.when(i == 0)
    def _init():
        loss_ref[...] = lv

    @pl.when(i > 0)
    def _acc():
        loss_ref[...] += lv

    @pl.when(i == nblk - 1)
    def _fin():
        loss_ref[...] = loss_ref[...] * scale


def _gates_to_hc(gates, c):
    i_g = jax.nn.sigmoid(gates[:, :HID])
    f_g = jax.nn.sigmoid(gates[:, HID:2 * HID])
    g_g = jnp.tanh(gates[:, 2 * HID:3 * HID])
    o_g = jax.nn.sigmoid(gates[:, 3 * HID:])
    c2 = f_g * c + i_g * g_g
    return o_g * jnp.tanh(c2), c2


def _lstm_kernel(x_ref, wih_ref, whh_ref, b_ref, o_ref, xw_ref, h_ref, c_ref,
                 *, bt, bn):
    i = pl.program_id(0)

    @pl.when(i == 0)
    def _init():
        h_ref[...] = jnp.zeros_like(h_ref)
        c_ref[...] = jnp.zeros_like(c_ref)

    xw_ref[...] = _dot_t(x_ref[...], wih_ref[...]) + b_ref[...]

    def step(t, _):
        gates = xw_ref[pl.ds(t * bn, bn), :] + _dot_t(h_ref[...], whh_ref[...])
        h2, c2 = _gates_to_hc(gates, c_ref[...])
        c_ref[...] = c2
        h_ref[...] = h2
        o_ref[pl.ds(t * bn, bn), :] = h2
        return 0

    jax.lax.fori_loop(0, bt, step, 0, unroll=10)


def _lstm_proj_kernel(x_ref, wih_ref, whh_ref, b_ref, outw_ref, outb_ref,
                      o_ref, xw_ref, hbuf_ref, h_ref, c_ref, *, bt, bn):
    i = pl.program_id(0)

    @pl.when(i == 0)
    def _init():
        h_ref[...] = jnp.zeros_like(h_ref)
        c_ref[...] = jnp.zeros_like(c_ref)

    xw_ref[...] = _dot_t(x_ref[...], wih_ref[...]) + b_ref[...]

    def step(t, _):
        gates = xw_ref[pl.ds(t * bn, bn), :] + _dot_t(h_ref[...], whh_ref[...])
        h2, c2 = _gates_to_hc(gates, c_ref[...])
        c_ref[...] = c2
        h_ref[...] = h2
        hbuf_ref[pl.ds(t * bn, bn), :] = h2
        return 0

    jax.lax.fori_loop(0, bt, step, 0, unroll=10)

    o_ref[...] = _dot_t(hbuf_ref[...], outw_ref[...]) + outb_ref[...]


def kernel(waveform, enc_W, enc_b, ln_g, ln_b, codebooks, Wih0, Whh0, bih0,
           bhh0, Wih1, Whh1, bih1, bhh1, out_W, out_b):
    Bn, T = waveform.shape
    frames = T // STRIDE
    rows = Bn * frames
    rb = 1000 if rows % 1000 == 0 else rows

    x = waveform.reshape(rows, STRIDE)

    nblk = rows // rb
    scale = 1.0 / (2.0 * rows * HID)
    quant, loss = pl.pallas_call(
        functools.partial(_encvq_kernel, nblk=nblk, scale=scale),
        grid=(nblk,),
        in_specs=[
            pl.BlockSpec((rb, STRIDE), lambda i: (i, 0)),
            pl.BlockSpec((HID, STRIDE), lambda i: (0, 0)),
            pl.BlockSpec((1, HID), lambda i: (0, 0)),
            pl.BlockSpec((1, HID), lambda i: (0, 0)),
            pl.BlockSpec((1, HID), lambda i: (0, 0)),
            pl.BlockSpec((NQ, CB, HID), lambda i: (0, 0, 0)),
        ],
        out_specs=[
            pl.BlockSpec((rb, HID), lambda i: (i, 0)),
            pl.BlockSpec((1, 1), lambda i: (0, 0)),
        ],
        out_shape=[
            jax.ShapeDtypeStruct((rows, HID), jnp.float32),
            jax.ShapeDtypeStruct((1, 1), jnp.float32),
        ],
    )(x, enc_W, enc_b.reshape(1, HID), ln_g.reshape(1, HID),
      ln_b.reshape(1, HID), codebooks)

    dec_in = (quant.reshape(Bn, frames, HID).swapaxes(0, 1)
              .reshape(rows, HID))

    bt_blk = 50 if frames % 50 == 0 else frames
    rbt = bt_blk * Bn
    b0 = (bih0 + bhh0).reshape(1, 4 * HID)
    b1 = (bih1 + bhh1).reshape(1, 4 * HID)

    h0 = pl.pallas_call(
        functools.partial(_lstm_kernel, bt=bt_blk, bn=Bn),
        grid=(frames // bt_blk,),
        in_specs=[
            pl.BlockSpec((rbt, HID), lambda i: (i, 0)),
            pl.BlockSpec((4 * HID, HID), lambda i: (0, 0)),
            pl.BlockSpec((4 * HID, HID), lambda i: (0, 0)),
            pl.BlockSpec((1, 4 * HID), lambda i: (0, 0)),
        ],
        out_specs=pl.BlockSpec((rbt, HID), lambda i: (i, 0)),
        out_shape=jax.ShapeDtypeStruct((rows, HID), jnp.float32),
        scratch_shapes=[
            pltpu.VMEM((rbt, 4 * HID), jnp.float32),
            pltpu.VMEM((Bn, HID), jnp.float32),
            pltpu.VMEM((Bn, HID), jnp.float32),
        ],
    )(dec_in, Wih0, Whh0, b0)

    out_flat = pl.pallas_call(
        functools.partial(_lstm_proj_kernel, bt=bt_blk, bn=Bn),
        grid=(frames // bt_blk,),
        in_specs=[
            pl.BlockSpec((rbt, HID), lambda i: (i, 0)),
            pl.BlockSpec((4 * HID, HID), lambda i: (0, 0)),
            pl.BlockSpec((4 * HID, HID), lambda i: (0, 0)),
            pl.BlockSpec((1, 4 * HID), lambda i: (0, 0)),
            pl.BlockSpec((STRIDE, HID), lambda i: (0, 0)),
            pl.BlockSpec((1, STRIDE), lambda i: (0, 0)),
        ],
        out_specs=pl.BlockSpec((rbt, STRIDE), lambda i: (i, 0)),
        out_shape=jax.ShapeDtypeStruct((rows, STRIDE), jnp.float32),
        scratch_shapes=[
            pltpu.VMEM((rbt, 4 * HID), jnp.float32),
            pltpu.VMEM((rbt, HID), jnp.float32),
            pltpu.VMEM((Bn, HID), jnp.float32),
            pltpu.VMEM((Bn, HID), jnp.float32),
        ],
    )(h0, Wih1, Whh1, b1, out_W, out_b.reshape(1, STRIDE))

    out = (out_flat.reshape(frames, Bn, STRIDE).swapaxes(0, 1)
           .reshape(Bn, frames * STRIDE))
    return out, loss.reshape(())


# speedup vs baseline: 1.2768x; 1.0178x over previous
import functools

import jax
import jax.numpy as jnp
from jax.experimental import pallas as pl
from jax.experimental.pallas import tpu as pltpu

STRIDE = 320
HID = 512
CB = 1024
NQ = 4


def _dot_t(a, b):
    return jax.lax.dot_general(a, b, (((1,), (1,)), ((), ())),
                               preferred_element_type=jnp.float32)


def _encvq_kernel(x_ref, w_ref, b_ref, g_ref, beta_ref, cb_ref, q_ref,
                  loss_ref, *, nblk, scale):
    i = pl.program_id(0)
    y = _dot_t(x_ref[...], w_ref[...]) + b_ref[...]
    m = jnp.mean(y, axis=-1, keepdims=True)
    v = jnp.mean((y - m) ** 2, axis=-1, keepdims=True)
    yn = (y - m) * jax.lax.rsqrt(v + 1e-5) * g_ref[...] + beta_ref[...]
    res = jnp.maximum(yn, 0.0)

    quant = jnp.zeros_like(res)
    loss = jnp.float32(0.0)
    for qi in range(NQ):
        cb = cb_ref[qi]
        d = (jnp.sum(res * res, axis=-1, keepdims=True)
             - 2.0 * _dot_t(res, cb)
             + jnp.sum(cb * cb, axis=-1)[None, :])
        idx = jnp.argmin(d, axis=-1)
        onehot = (jax.lax.broadcasted_iota(jnp.int32, d.shape, 1)
                  == idx[:, None]).astype(jnp.float32)
        q = jax.lax.dot_general(onehot, cb, (((1,), (0,)), ((), ())),
                                preferred_element_type=jnp.float32,
                                precision=jax.lax.Precision.HIGHEST)
        loss = loss + jnp.sum((q - res) ** 2)
        res = res - q
        quant = quant + q
    q_ref[...] = quant

    lv = loss.reshape(1, 1)

    @pl.when(i == 0)
    def _init():
        loss_ref[...] = lv

    @pl.when(i > 0)
    def _acc():
        loss_ref[...] += lv

    @pl.when(i == nblk - 1)
    def _fin():
        loss_ref[...] = loss_ref[...] * scale


def _gates_to_hc(gates, c):
    i_g = jax.nn.sigmoid(gates[:, :HID])
    f_g = jax.nn.sigmoid(gates[:, HID:2 * HID])
    g_g = jnp.tanh(gates[:, 2 * HID:3 * HID])
    o_g = jax.nn.sigmoid(gates[:, 3 * HID:])
    c2 = f_g * c + i_g * g_g
    return o_g * jnp.tanh(c2), c2


def _lstm_kernel(x_ref, wih_ref, whh_ref, b_ref, o_ref, xw_ref, h_ref, c_ref,
                 *, bt, bn):
    i = pl.program_id(0)

    @pl.when(i == 0)
    def _init():
        h_ref[...] = jnp.zeros_like(h_ref)
        c_ref[...] = jnp.zeros_like(c_ref)

    xw_ref[...] = _dot_t(x_ref[...], wih_ref[...]) + b_ref[...]

    def step(t, _):
        gates = xw_ref[pl.ds(t * bn, bn), :] + _dot_t(h_ref[...], whh_ref[...])
        h2, c2 = _gates_to_hc(gates, c_ref[...])
        c_ref[...] = c2
        h_ref[...] = h2
        o_ref[pl.ds(t * bn, bn), :] = h2
        return 0

    jax.lax.fori_loop(0, bt, step, 0, unroll=25)


def _lstm_proj_kernel(x_ref, wih_ref, whh_ref, b_ref, outw_ref, outb_ref,
                      o_ref, xw_ref, hbuf_ref, h_ref, c_ref, *, bt, bn):
    i = pl.program_id(0)

    @pl.when(i == 0)
    def _init():
        h_ref[...] = jnp.zeros_like(h_ref)
        c_ref[...] = jnp.zeros_like(c_ref)

    xw_ref[...] = _dot_t(x_ref[...], wih_ref[...]) + b_ref[...]

    def step(t, _):
        gates = xw_ref[pl.ds(t * bn, bn), :] + _dot_t(h_ref[...], whh_ref[...])
        h2, c2 = _gates_to_hc(gates, c_ref[...])
        c_ref[...] = c2
        h_ref[...] = h2
        hbuf_ref[pl.ds(t * bn, bn), :] = h2
        return 0

    jax.lax.fori_loop(0, bt, step, 0, unroll=25)

    o_ref[...] = _dot_t(hbuf_ref[...], outw_ref[...]) + outb_ref[...]


def kernel(waveform, enc_W, enc_b, ln_g, ln_b, codebooks, Wih0, Whh0, bih0,
           bhh0, Wih1, Whh1, bih1, bhh1, out_W, out_b):
    Bn, T = waveform.shape
    frames = T // STRIDE
    rows = Bn * frames
    rb = 1000 if rows % 1000 == 0 else rows

    x = waveform.reshape(rows, STRIDE)

    nblk = rows // rb
    scale = 1.0 / (2.0 * rows * HID)
    quant, loss = pl.pallas_call(
        functools.partial(_encvq_kernel, nblk=nblk, scale=scale),
        grid=(nblk,),
        in_specs=[
            pl.BlockSpec((rb, STRIDE), lambda i: (i, 0)),
            pl.BlockSpec((HID, STRIDE), lambda i: (0, 0)),
            pl.BlockSpec((1, HID), lambda i: (0, 0)),
            pl.BlockSpec((1, HID), lambda i: (0, 0)),
            pl.BlockSpec((1, HID), lambda i: (0, 0)),
            pl.BlockSpec((NQ, CB, HID), lambda i: (0, 0, 0)),
        ],
        out_specs=[
            pl.BlockSpec((rb, HID), lambda i: (i, 0)),
            pl.BlockSpec((1, 1), lambda i: (0, 0)),
        ],
        out_shape=[
            jax.ShapeDtypeStruct((rows, HID), jnp.float32),
            jax.ShapeDtypeStruct((1, 1), jnp.float32),
        ],
    )(x, enc_W, enc_b.reshape(1, HID), ln_g.reshape(1, HID),
      ln_b.reshape(1, HID), codebooks)

    dec_in = (quant.reshape(Bn, frames, HID).swapaxes(0, 1)
              .reshape(rows, HID))

    bt_blk = 50 if frames % 50 == 0 else frames
    rbt = bt_blk * Bn
    b0 = (bih0 + bhh0).reshape(1, 4 * HID)
    b1 = (bih1 + bhh1).reshape(1, 4 * HID)

    h0 = pl.pallas_call(
        functools.partial(_lstm_kernel, bt=bt_blk, bn=Bn),
        grid=(frames // bt_blk,),
        in_specs=[
            pl.BlockSpec((rbt, HID), lambda i: (i, 0)),
            pl.BlockSpec((4 * HID, HID), lambda i: (0, 0)),
            pl.BlockSpec((4 * HID, HID), lambda i: (0, 0)),
            pl.BlockSpec((1, 4 * HID), lambda i: (0, 0)),
        ],
        out_specs=pl.BlockSpec((rbt, HID), lambda i: (i, 0)),
        out_shape=jax.ShapeDtypeStruct((rows, HID), jnp.float32),
        scratch_shapes=[
            pltpu.VMEM((rbt, 4 * HID), jnp.float32),
            pltpu.VMEM((Bn, HID), jnp.float32),
            pltpu.VMEM((Bn, HID), jnp.float32),
        ],
    )(dec_in, Wih0, Whh0, b0)

    out_flat = pl.pallas_call(
        functools.partial(_lstm_proj_kernel, bt=bt_blk, bn=Bn),
        grid=(frames // bt_blk,),
        in_specs=[
            pl.BlockSpec((rbt, HID), lambda i: (i, 0)),
            pl.BlockSpec((4 * HID, HID), lambda i: (0, 0)),
            pl.BlockSpec((4 * HID, HID), lambda i: (0, 0)),
            pl.BlockSpec((1, 4 * HID), lambda i: (0, 0)),
            pl.BlockSpec((STRIDE, HID), lambda i: (0, 0)),
            pl.BlockSpec((1, STRIDE), lambda i: (0, 0)),
        ],
        out_specs=pl.BlockSpec((rbt, STRIDE), lambda i: (i, 0)),
        out_shape=jax.ShapeDtypeStruct((rows, STRIDE), jnp.float32),
        scratch_shapes=[
            pltpu.VMEM((rbt, 4 * HID), jnp.float32),
            pltpu.VMEM((rbt, HID), jnp.float32),
            pltpu.VMEM((Bn, HID), jnp.float32),
            pltpu.VMEM((Bn, HID), jnp.float32),
        ],
    )(h0, Wih1, Whh1, b1, out_W, out_b.reshape(1, STRIDE))

    out = (out_flat.reshape(frames, Bn, STRIDE).swapaxes(0, 1)
           .reshape(Bn, frames * STRIDE))
    return out, loss.reshape(())
